# baseline (speedup 1.0000x reference)
import functools

import jax
import jax.numpy as jnp
from jax import lax
from jax.experimental import pallas as pl
from jax.experimental.pallas import tpu as pltpu
from jax.experimental.pallas import tpu_sc as plsc

_NC = 2
_NS = 16
_NW = _NC * _NS

_D = 128
_C = 64


@functools.partial(jax.jit, static_argnames=("n_chunks",))
def _gather_rows(idx, table, *, n_chunks):
    b_per_w = n_chunks * _C
    total = _NW * b_per_w
    mesh = plsc.VectorSubcoreMesh(core_axis_name="c", subcore_axis_name="s")

    nb = 10
    assert n_chunks % nb == 0

    @functools.partial(
        pl.kernel,
        mesh=mesh,
        out_type=jax.ShapeDtypeStruct((total, _D), jnp.float32),
        scratch_types=[
            pltpu.VMEM((n_chunks, _C), jnp.int32),
            pltpu.VMEM((nb, _C, _D), jnp.float32),
            [pltpu.SemaphoreType.DMA] * nb,
            [pltpu.SemaphoreType.DMA] * nb,
        ],
    )
    def k(idx_hbm, table_hbm, out_hbm, idx_v, rows_v, gsem, wsem):
        wid = lax.axis_index("s") * _NC + lax.axis_index("c")
        base = wid * b_per_w
        pltpu.sync_copy(idx_hbm.at[wid], idx_v)

        def gather(j, b):
            pltpu.async_copy(table_hbm.at[idx_v.at[j]], rows_v.at[b], gsem[b])

        def wait_gather(j, b):
            pltpu.make_async_copy(
                table_hbm.at[idx_v.at[j]], rows_v.at[b], gsem[b]).wait()

        def write(j, b):
            pltpu.async_copy(
                rows_v.at[b], out_hbm.at[pl.ds(base + j * _C, _C)], wsem[b])

        def wait_write(j, b):
            pltpu.make_async_copy(
                rows_v.at[b], out_hbm.at[pl.ds(base + j * _C, _C)],
                wsem[b]).wait()

        for t in range(nb - 1):
            gather(t, t)

        def body(m, carry):
            for t in range(nb):
                j = nb * m + t
                bn = (t + nb - 1) % nb
                if t == 0:
                    @pl.when(m > 0)
                    def _():
                        wait_write(j - 1, bn)
                    gather(j + nb - 1, bn)
                else:
                    wait_write(j - 1, bn)

                    @pl.when(j + nb - 1 < n_chunks)
                    def _():
                        gather(j + nb - 1, bn)
                wait_gather(j, t)
                write(j, t)
            return carry

        lax.fori_loop(0, n_chunks // nb, body, 0)
        wait_write(n_chunks - 1, (n_chunks - 1) % nb)

    return k(idx, table)


def kernel(speaker_ids, table):
    b0, s = speaker_ids.shape
    total = b0 * s
    n_chunks = total // (_NW * _C)
    idx = speaker_ids.astype(jnp.int32).reshape(_NW, n_chunks, _C)
    out = _gather_rows(idx, table, n_chunks=n_chunks)
    return out.reshape(b0, s, _D)


# speedup vs baseline: 1.0153x; 1.0153x over previous
import functools

import jax
import jax.numpy as jnp
from jax import lax
from jax.experimental import pallas as pl
from jax.experimental.pallas import tpu as pltpu
from jax.experimental.pallas import tpu_sc as plsc

_NC = 2
_NS = 16
_NW = _NC * _NS

_D = 128
_C = 128


@functools.partial(jax.jit, static_argnames=("n_chunks",))
def _gather_rows(idx, table, *, n_chunks):
    b_per_w = n_chunks * _C
    total = _NW * b_per_w
    mesh = plsc.VectorSubcoreMesh(core_axis_name="c", subcore_axis_name="s")

    nb = 5
    assert n_chunks % nb == 0

    @functools.partial(
        pl.kernel,
        mesh=mesh,
        out_type=jax.ShapeDtypeStruct((total, _D), jnp.float32),
        scratch_types=[
            pltpu.VMEM((n_chunks, _C), jnp.int32),
            pltpu.VMEM((nb, _C, _D), jnp.float32),
            [pltpu.SemaphoreType.DMA] * nb,
            [pltpu.SemaphoreType.DMA] * nb,
        ],
    )
    def k(idx_hbm, table_hbm, out_hbm, idx_v, rows_v, gsem, wsem):
        wid = lax.axis_index("s") * _NC + lax.axis_index("c")
        base = wid * b_per_w
        pltpu.sync_copy(idx_hbm.at[wid], idx_v)

        def gather(j, b):
            pltpu.async_copy(table_hbm.at[idx_v.at[j]], rows_v.at[b], gsem[b])

        def wait_gather(j, b):
            pltpu.make_async_copy(
                table_hbm.at[idx_v.at[j]], rows_v.at[b], gsem[b]).wait()

        def write(j, b):
            pltpu.async_copy(
                rows_v.at[b], out_hbm.at[pl.ds(base + j * _C, _C)], wsem[b])

        def wait_write(j, b):
            pltpu.make_async_copy(
                rows_v.at[b], out_hbm.at[pl.ds(base + j * _C, _C)],
                wsem[b]).wait()

        for t in range(nb - 1):
            gather(t, t)

        def body(m, carry):
            for t in range(nb):
                j = nb * m + t
                bn = (t + nb - 1) % nb
                if t == 0:
                    @pl.when(m > 0)
                    def _():
                        wait_write(j - 1, bn)
                    gather(j + nb - 1, bn)
                else:
                    wait_write(j - 1, bn)

                    @pl.when(j + nb - 1 < n_chunks)
                    def _():
                        gather(j + nb - 1, bn)
                wait_gather(j, t)
                write(j, t)
            return carry

        lax.fori_loop(0, n_chunks // nb, body, 0)
        wait_write(n_chunks - 1, (n_chunks - 1) % nb)

    return k(idx, table)


def kernel(speaker_ids, table):
    b0, s = speaker_ids.shape
    total = b0 * s
    n_chunks = total // (_NW * _C)
    idx = speaker_ids.astype(jnp.int32).reshape(_NW, n_chunks, _C)
    out = _gather_rows(idx, table, n_chunks=n_chunks)
    return out.reshape(b0, s, _D)
